# Initial kernel scaffold; baseline (speedup 1.0000x reference)
import jax
import jax.numpy as jnp
from jax.experimental import pallas as pl


def kernel(user_ids, input_seq, target_item, item_table, user_table, W1, b1, gamma, beta, W2, b2):
    raise NotImplementedError("write your pallas kernel here")


# speedup vs baseline: 1.2357x; 1.2357x over previous
import functools

import jax
import jax.numpy as jnp
from jax import lax
from jax.experimental import pallas as pl
from jax.experimental.pallas import tpu as pltpu
from jax.experimental.pallas import tpu_sc as plsc

_B = 4096
_L = 200
_D = 32
_NC = 2
_NS = 16
_NW = _NC * _NS
_BPW = _B // _NW


def _sc_gather_pool(seq_flat, tgt, uid, item_table, user_table):
    mesh = plsc.VectorSubcoreMesh(core_axis_name="c", subcore_axis_name="s")

    @functools.partial(
        pl.kernel,
        out_type=(
            jax.ShapeDtypeStruct((_B, _D), jnp.float32),
            jax.ShapeDtypeStruct((_B, _D), jnp.float32),
            jax.ShapeDtypeStruct((_B, _D), jnp.float32),
        ),
        mesh=mesh,
        compiler_params=pltpu.CompilerParams(use_tc_tiling_on_sc=False),
        scratch_types=(
            pltpu.VMEM((_BPW * _L,), jnp.int32),
            pltpu.VMEM((_L, _D), jnp.float32),
            pltpu.VMEM((_BPW, _D), jnp.float32),
            pltpu.VMEM((_BPW,), jnp.int32),
            pltpu.VMEM((_BPW, _D), jnp.float32),
            pltpu.VMEM((_BPW,), jnp.int32),
            pltpu.VMEM((_BPW, _D), jnp.float32),
            pltpu.SemaphoreType.DMA,
            pltpu.SemaphoreType.DMA,
            pltpu.SemaphoreType.DMA,
        ),
    )
    def k(seq_ref, tgt_ref, uid_ref, item_ref, user_ref,
          pooled_out, trow_out, urow_out,
          idx_v, rows_v, pooled_v, tidx_v, trows_v, uidx_v, urows_v,
          sem_seq, sem_t, sem_u):
        wid = lax.axis_index("s") * _NC + lax.axis_index("c")
        base = pl.multiple_of(wid * _BPW, _BPW)

        pltpu.sync_copy(seq_ref.at[pl.ds(base * _L, _BPW * _L)], idx_v)
        pltpu.sync_copy(tgt_ref.at[pl.ds(base, _BPW)], tidx_v)
        pltpu.sync_copy(uid_ref.at[pl.ds(base, _BPW)], uidx_v)
        tcp = pltpu.async_copy(item_ref.at[tidx_v], trows_v, sem_t)
        ucp = pltpu.async_copy(user_ref.at[uidx_v], urows_v, sem_u)

        inv_l = jnp.float32(1.0 / _L)

        def row_body(r, carry):
            off = pl.multiple_of(r * _L, 8)
            cp1 = pltpu.async_copy(
                item_ref.at[idx_v.at[pl.ds(off, 96)]],
                rows_v.at[pl.ds(0, 96)], sem_seq)
            cp2 = pltpu.async_copy(
                item_ref.at[idx_v.at[pl.ds(off + 96, 104)]],
                rows_v.at[pl.ds(96, 104)], sem_seq)
            cp1.wait()
            cp2.wait()

            def acc(j, a):
                return (a[0] + rows_v[j, pl.ds(0, 16)],
                        a[1] + rows_v[j, pl.ds(16, 16)])

            z = jnp.zeros((16,), jnp.float32)
            a0, a1 = lax.fori_loop(0, _L, acc, (z, z))
            pooled_v[r, pl.ds(0, 16)] = a0 * inv_l
            pooled_v[r, pl.ds(16, 16)] = a1 * inv_l
            return carry

        lax.fori_loop(0, _BPW, row_body, 0)

        tcp.wait()
        ucp.wait()
        pltpu.sync_copy(pooled_v, pooled_out.at[pl.ds(base, _BPW), :])
        pltpu.sync_copy(trows_v, trow_out.at[pl.ds(base, _BPW), :])
        pltpu.sync_copy(urows_v, urow_out.at[pl.ds(base, _BPW), :])

    return k(seq_flat, tgt, uid, item_table, user_table)


def _tc_head(pooled, trow, urow, W1, b1, gamma, beta, W2, b2):
    def body(p_ref, t_ref, u_ref, w1_ref, b1_ref, g_ref, be_ref,
             w2_ref, b2_ref, o_ref):
        w1 = w1_ref[...]
        dn = (((1,), (1,)), ((), ()))
        h = lax.dot_general(p_ref[...], w1[:, 0:32], dn,
                            precision=lax.Precision.HIGHEST)
        h = h + lax.dot_general(t_ref[...], w1[:, 32:64], dn,
                                precision=lax.Precision.HIGHEST)
        h = h + lax.dot_general(u_ref[...], w1[:, 64:96], dn,
                                precision=lax.Precision.HIGHEST)
        h = h + b1_ref[...]
        mu = jnp.mean(h, axis=0, keepdims=True)
        d = h - mu
        var = jnp.mean(d * d, axis=0, keepdims=True)
        hn = d * lax.rsqrt(var + 1e-5) * g_ref[...] + be_ref[...]
        hr = jnp.maximum(hn, 0.0)
        out = jnp.sum(hr * w2_ref[...], axis=1, keepdims=True) + b2_ref[...]
        o_ref[...] = 1.0 / (1.0 + jnp.exp(-out))

    return pl.pallas_call(
        body,
        out_shape=jax.ShapeDtypeStruct((_B, 1), jnp.float32),
    )(pooled, trow, urow, W1, b1, gamma, beta, W2, b2)


def kernel(user_ids, input_seq, target_item, item_table, user_table,
           W1, b1, gamma, beta, W2, b2):
    seq_flat = input_seq.reshape(-1)
    pooled, trow, urow = _sc_gather_pool(
        seq_flat, target_item, user_ids, item_table, user_table)
    return _tc_head(pooled, trow, urow, W1, b1, gamma, beta, W2, b2)


# speedup vs baseline: 1.3919x; 1.1264x over previous
import functools

import jax
import jax.numpy as jnp
from jax import lax
from jax.experimental import pallas as pl
from jax.experimental.pallas import tpu as pltpu
from jax.experimental.pallas import tpu_sc as plsc

_B = 4096
_L = 200
_D = 32
_NC = 2
_NS = 16
_NW = _NC * _NS
_BPW = _B // _NW


def _sc_gather_pool(seq_flat, tgt, uid, item_table, user_table):
    mesh = plsc.VectorSubcoreMesh(core_axis_name="c", subcore_axis_name="s")

    @functools.partial(
        pl.kernel,
        out_type=(
            jax.ShapeDtypeStruct((_B, _D), jnp.float32),
            jax.ShapeDtypeStruct((_B, _D), jnp.float32),
            jax.ShapeDtypeStruct((_B, _D), jnp.float32),
        ),
        mesh=mesh,
        compiler_params=pltpu.CompilerParams(use_tc_tiling_on_sc=False),
        scratch_types=(
            pltpu.VMEM((_BPW * _L,), jnp.int32),
            pltpu.VMEM((_L, _D), jnp.float32),
            pltpu.VMEM((_L, _D), jnp.float32),
            pltpu.VMEM((_BPW, _D), jnp.float32),
            pltpu.VMEM((_BPW,), jnp.int32),
            pltpu.VMEM((_BPW, _D), jnp.float32),
            pltpu.VMEM((_BPW,), jnp.int32),
            pltpu.VMEM((_BPW, _D), jnp.float32),
            pltpu.SemaphoreType.DMA,
            pltpu.SemaphoreType.DMA,
            pltpu.SemaphoreType.DMA,
            pltpu.SemaphoreType.DMA,
        ),
    )
    def k(seq_ref, tgt_ref, uid_ref, item_ref, user_ref,
          pooled_out, trow_out, urow_out,
          idx_v, rows0_v, rows1_v, pooled_v, tidx_v, trows_v, uidx_v, urows_v,
          sem0, sem1, sem_t, sem_u):
        wid = lax.axis_index("s") * _NC + lax.axis_index("c")
        base = pl.multiple_of(wid * _BPW, _BPW)

        pltpu.sync_copy(seq_ref.at[pl.ds(base * _L, _BPW * _L)], idx_v)
        pltpu.sync_copy(tgt_ref.at[pl.ds(base, _BPW)], tidx_v)
        pltpu.sync_copy(uid_ref.at[pl.ds(base, _BPW)], uidx_v)
        tcp = pltpu.async_copy(item_ref.at[tidx_v], trows_v, sem_t)
        ucp = pltpu.async_copy(user_ref.at[uidx_v], urows_v, sem_u)

        inv_l = jnp.float32(1.0 / _L)

        def issue(row, buf, sem):
            off = pl.multiple_of(row * _L, 8)
            pltpu.async_copy(item_ref.at[idx_v.at[pl.ds(off, 96)]],
                             buf.at[pl.ds(0, 96)], sem)
            pltpu.async_copy(item_ref.at[idx_v.at[pl.ds(off + 96, 104)]],
                             buf.at[pl.ds(96, 104)], sem)

        def wait_buf(buf, sem):
            pltpu.make_async_copy(item_ref.at[pl.ds(0, _L), :], buf, sem).wait()

        def accumulate(row, buf):
            def acc8(j2, a):
                a0, a1, b0, b1 = a
                j = pl.multiple_of(j2 * 8, 8)
                for t in range(8):
                    lo = buf[j + t, pl.ds(0, 16)]
                    hi = buf[j + t, pl.ds(16, 16)]
                    if t % 2 == 0:
                        a0 = a0 + lo
                        a1 = a1 + hi
                    else:
                        b0 = b0 + lo
                        b1 = b1 + hi
                return (a0, a1, b0, b1)

            z = jnp.zeros((16,), jnp.float32)
            a0, a1, b0, b1 = lax.fori_loop(0, _L // 8, acc8, (z, z, z, z))
            pooled_v[row, pl.ds(0, 16)] = (a0 + b0) * inv_l
            pooled_v[row, pl.ds(16, 16)] = (a1 + b1) * inv_l

        issue(0, rows0_v, sem0)
        issue(1, rows1_v, sem1)

        def pair_body(r2, carry):
            r = r2 * 2
            wait_buf(rows0_v, sem0)
            accumulate(r, rows0_v)

            @pl.when(r2 < _BPW // 2 - 1)
            def _():
                issue(r + 2, rows0_v, sem0)

            wait_buf(rows1_v, sem1)
            accumulate(r + 1, rows1_v)

            @pl.when(r2 < _BPW // 2 - 1)
            def _():
                issue(r + 3, rows1_v, sem1)

            return carry

        lax.fori_loop(0, _BPW // 2, pair_body, 0)

        tcp.wait()
        ucp.wait()
        pltpu.sync_copy(pooled_v, pooled_out.at[pl.ds(base, _BPW), :])
        pltpu.sync_copy(trows_v, trow_out.at[pl.ds(base, _BPW), :])
        pltpu.sync_copy(urows_v, urow_out.at[pl.ds(base, _BPW), :])

    return k(seq_flat, tgt, uid, item_table, user_table)


def _tc_head(pooled, trow, urow, W1, b1, gamma, beta, W2, b2):
    def body(p_ref, t_ref, u_ref, w1_ref, b1_ref, g_ref, be_ref,
             w2_ref, b2_ref, o_ref):
        w1 = w1_ref[...]
        dn = (((1,), (1,)), ((), ()))
        h = lax.dot_general(p_ref[...], w1[:, 0:32], dn,
                            precision=lax.Precision.HIGHEST)
        h = h + lax.dot_general(t_ref[...], w1[:, 32:64], dn,
                                precision=lax.Precision.HIGHEST)
        h = h + lax.dot_general(u_ref[...], w1[:, 64:96], dn,
                                precision=lax.Precision.HIGHEST)
        h = h + b1_ref[...]
        mu = jnp.mean(h, axis=0, keepdims=True)
        d = h - mu
        var = jnp.mean(d * d, axis=0, keepdims=True)
        hn = d * lax.rsqrt(var + 1e-5) * g_ref[...] + be_ref[...]
        hr = jnp.maximum(hn, 0.0)
        out = jnp.sum(hr * w2_ref[...], axis=1, keepdims=True) + b2_ref[...]
        o_ref[...] = 1.0 / (1.0 + jnp.exp(-out))

    return pl.pallas_call(
        body,
        out_shape=jax.ShapeDtypeStruct((_B, 1), jnp.float32),
    )(pooled, trow, urow, W1, b1, gamma, beta, W2, b2)


def kernel(user_ids, input_seq, target_item, item_table, user_table,
           W1, b1, gamma, beta, W2, b2):
    seq_flat = input_seq.reshape(-1)
    pooled, trow, urow = _sc_gather_pool(
        seq_flat, target_item, user_ids, item_table, user_table)
    return _tc_head(pooled, trow, urow, W1, b1, gamma, beta, W2, b2)


# speedup vs baseline: 1.3949x; 1.0022x over previous
import functools

import jax
import jax.numpy as jnp
from jax import lax
from jax.experimental import pallas as pl
from jax.experimental.pallas import tpu as pltpu
from jax.experimental.pallas import tpu_sc as plsc

_B = 4096
_L = 200
_D = 32
_NC = 2
_NS = 16
_NW = _NC * _NS
_BPW = _B // _NW


def _sc_gather_pool(seq_flat, tgt, uid, item_table, user_table):
    mesh = plsc.VectorSubcoreMesh(core_axis_name="c", subcore_axis_name="s")

    @functools.partial(
        pl.kernel,
        out_type=(
            jax.ShapeDtypeStruct((_B, _D), jnp.float32),
            jax.ShapeDtypeStruct((_B, _D), jnp.float32),
            jax.ShapeDtypeStruct((_B, _D), jnp.float32),
        ),
        mesh=mesh,
        compiler_params=pltpu.CompilerParams(use_tc_tiling_on_sc=False),
        scratch_types=(
            pltpu.VMEM((_BPW, _L), jnp.int32),
            pltpu.VMEM((_L, _D), jnp.float32),
            pltpu.VMEM((_L, _D), jnp.float32),
            pltpu.VMEM((_BPW, _D), jnp.float32),
            pltpu.VMEM((_BPW,), jnp.int32),
            pltpu.VMEM((_BPW, _D), jnp.float32),
            pltpu.VMEM((_BPW,), jnp.int32),
            pltpu.VMEM((_BPW, _D), jnp.float32),
            pltpu.SemaphoreType.DMA,
            pltpu.SemaphoreType.DMA,
            pltpu.SemaphoreType.DMA,
            pltpu.SemaphoreType.DMA,
        ),
    )
    def k(seq_ref, tgt_ref, uid_ref, item_ref, user_ref,
          pooled_out, trow_out, urow_out,
          idx_v, rows0_v, rows1_v, pooled_v, tidx_v, trows_v, uidx_v, urows_v,
          sem0, sem1, sem_t, sem_u):
        wid = lax.axis_index("s") * _NC + lax.axis_index("c")
        base = pl.multiple_of(wid * _BPW, _BPW)

        pltpu.sync_copy(seq_ref.at[pl.ds(base, _BPW), :], idx_v)
        pltpu.sync_copy(tgt_ref.at[pl.ds(base, _BPW)], tidx_v)
        pltpu.sync_copy(uid_ref.at[pl.ds(base, _BPW)], uidx_v)
        tcp = pltpu.async_copy(item_ref.at[tidx_v], trows_v, sem_t)
        ucp = pltpu.async_copy(user_ref.at[uidx_v], urows_v, sem_u)

        inv_l = jnp.float32(1.0 / _L)

        def issue(row, buf, sem):
            pltpu.async_copy(item_ref.at[idx_v.at[row, pl.ds(0, 96)]],
                             buf.at[pl.ds(0, 96)], sem)
            pltpu.async_copy(item_ref.at[idx_v.at[row, pl.ds(96, 104)]],
                             buf.at[pl.ds(96, 104)], sem)

        def wait_buf(buf, sem):
            pltpu.make_async_copy(item_ref.at[pl.ds(0, _L), :], buf, sem).wait()

        def accumulate(row, buf):
            def acc8(j2, a):
                a0, a1, b0, b1 = a
                j = pl.multiple_of(j2 * 8, 8)
                for t in range(8):
                    lo = buf[j + t, pl.ds(0, 16)]
                    hi = buf[j + t, pl.ds(16, 16)]
                    if t % 2 == 0:
                        a0 = a0 + lo
                        a1 = a1 + hi
                    else:
                        b0 = b0 + lo
                        b1 = b1 + hi
                return (a0, a1, b0, b1)

            z = jnp.zeros((16,), jnp.float32)
            a0, a1, b0, b1 = lax.fori_loop(0, _L // 8, acc8, (z, z, z, z))
            pooled_v[row, pl.ds(0, 16)] = (a0 + b0) * inv_l
            pooled_v[row, pl.ds(16, 16)] = (a1 + b1) * inv_l

        issue(0, rows0_v, sem0)
        issue(1, rows1_v, sem1)

        def pair_body(r2, carry):
            r = r2 * 2
            wait_buf(rows0_v, sem0)
            accumulate(r, rows0_v)

            @pl.when(r2 < _BPW // 2 - 1)
            def _():
                issue(r + 2, rows0_v, sem0)

            wait_buf(rows1_v, sem1)
            accumulate(r + 1, rows1_v)

            @pl.when(r2 < _BPW // 2 - 1)
            def _():
                issue(r + 3, rows1_v, sem1)

            return carry

        lax.fori_loop(0, _BPW // 2, pair_body, 0)

        tcp.wait()
        ucp.wait()
        pltpu.sync_copy(pooled_v, pooled_out.at[pl.ds(base, _BPW), :])
        pltpu.sync_copy(trows_v, trow_out.at[pl.ds(base, _BPW), :])
        pltpu.sync_copy(urows_v, urow_out.at[pl.ds(base, _BPW), :])

    return k(seq_flat, tgt, uid, item_table, user_table)


def _tc_head(pooled, trow, urow, W1, b1, gamma, beta, W2, b2):
    def body(p_ref, t_ref, u_ref, w1_ref, b1_ref, g_ref, be_ref,
             w2_ref, b2_ref, o_ref):
        w1 = w1_ref[...]
        dn = (((1,), (1,)), ((), ()))
        h = lax.dot_general(p_ref[...], w1[:, 0:32], dn,
                            precision=lax.Precision.HIGHEST)
        h = h + lax.dot_general(t_ref[...], w1[:, 32:64], dn,
                                precision=lax.Precision.HIGHEST)
        h = h + lax.dot_general(u_ref[...], w1[:, 64:96], dn,
                                precision=lax.Precision.HIGHEST)
        h = h + b1_ref[...]
        mu = jnp.mean(h, axis=0, keepdims=True)
        d = h - mu
        var = jnp.mean(d * d, axis=0, keepdims=True)
        hn = d * lax.rsqrt(var + 1e-5) * g_ref[...] + be_ref[...]
        hr = jnp.maximum(hn, 0.0)
        out = jnp.sum(hr * w2_ref[...], axis=1, keepdims=True) + b2_ref[...]
        o_ref[...] = 1.0 / (1.0 + jnp.exp(-out))

    return pl.pallas_call(
        body,
        out_shape=jax.ShapeDtypeStruct((_B, 1), jnp.float32),
    )(pooled, trow, urow, W1, b1, gamma, beta, W2, b2)


def kernel(user_ids, input_seq, target_item, item_table, user_table,
           W1, b1, gamma, beta, W2, b2):
    pooled, trow, urow = _sc_gather_pool(
        input_seq, target_item, user_ids, item_table, user_table)
    return _tc_head(pooled, trow, urow, W1, b1, gamma, beta, W2, b2)
